# baseline (speedup 1.0000x reference)
import functools

import jax
import jax.numpy as jnp
from jax import lax
from jax.experimental import pallas as pl
from jax.experimental.pallas import tpu as pltpu
from jax.experimental.pallas import tpu_sc as plsc

F_SP = 26
F_DN = 13
NF = F_SP + F_DN
D = 16
EPS = 1e-5

NC = 2
NS = 16
NW = NC * NS
CHUNK = 128
NG = CHUNK // 16


def _sc_gather_sum(inputs_flat, tables):
    b = inputs_flat.shape[0] // NF
    nchunk = b // (CHUNK * NW)
    mesh = plsc.VectorSubcoreMesh(core_axis_name="c", subcore_axis_name="s")

    @functools.partial(
        pl.kernel,
        out_type=jax.ShapeDtypeStruct((b, D), jnp.float32),
        mesh=mesh,
        scratch_types=[
            pltpu.VMEM((CHUNK * NF,), jnp.float32),
            pltpu.VMEM((F_SP, 1, CHUNK), jnp.int32),
            pltpu.VMEM((F_SP * CHUNK, D), jnp.float32),
            pltpu.VMEM((CHUNK, D), jnp.float32),
            pltpu.SemaphoreType.DMA,
        ],
        compiler_params=pltpu.CompilerParams(
            use_tc_tiling_on_sc=False, needs_layout_passes=False
        ),
    )
    def k(in_hbm, tab_hbm, out_hbm, raw_v, idx_v, rows_v, red_v, gsem):
        wid = lax.axis_index("c") * NS + lax.axis_index("s")
        lane = lax.iota(jnp.int32, 16)

        @pl.loop(0, nchunk)
        def _chunk(kc):
            grp = wid * nchunk + kc
            pltpu.sync_copy(in_hbm.at[pl.ds(grp * CHUNK * NF, CHUNK * NF)], raw_v)

            @pl.loop(0, F_SP)
            def _field(f):
                for g in range(NG):
                    pos = lane * NF + (g * 16 * NF) + f
                    ids = plsc.load_gather(raw_v, [pos]).astype(jnp.int32)
                    idx_v[f, 0, pl.ds(g * 16, 16)] = ids
                pltpu.async_copy(
                    tab_hbm.at[f].at[idx_v.at[f, 0]],
                    rows_v.at[pl.ds(f * CHUNK, CHUNK), :],
                    gsem,
                )

            pltpu.make_async_copy(
                tab_hbm.at[0, pl.ds(0, F_SP * CHUNK), :], rows_v, gsem
            ).wait()

            @pl.loop(0, CHUNK, unroll=2)
            def _row(i):
                a0 = rows_v[i]
                a1 = rows_v[CHUNK + i]
                a2 = rows_v[2 * CHUNK + i]
                a3 = rows_v[3 * CHUNK + i]
                for f in range(4, F_SP - 2, 4):
                    a0 = a0 + rows_v[f * CHUNK + i]
                    a1 = a1 + rows_v[(f + 1) * CHUNK + i]
                    a2 = a2 + rows_v[(f + 2) * CHUNK + i]
                    a3 = a3 + rows_v[(f + 3) * CHUNK + i]
                a0 = a0 + rows_v[(F_SP - 2) * CHUNK + i]
                a1 = a1 + rows_v[(F_SP - 1) * CHUNK + i]
                red_v[i] = (a0 + a1) + (a2 + a3)

            pltpu.sync_copy(red_v, out_hbm.at[pl.ds(grp * CHUNK, CHUNK), :])

    return k(inputs_flat, tables)


def _tc_combine(inputs, acc, gamma, beta, wt, bias):
    def body(in_ref, acc_ref, g_ref, b_ref, w_ref, bias_ref, out_ref):
        d = in_ref[:, F_SP:]
        mean = jnp.mean(d, axis=0, keepdims=True)
        c = d - mean
        var = jnp.mean(c * c, axis=0, keepdims=True)
        bn = c * lax.rsqrt(var + EPS) * g_ref[...][None, :] + b_ref[...][None, :]
        dense_logit = jnp.sum(bn * w_ref[...], axis=1, keepdims=True)
        sparse_logit = jnp.sum(acc_ref[...], axis=1, keepdims=True)
        out_ref[...] = sparse_logit + dense_logit + bias_ref[...][None, :]

    return pl.pallas_call(
        body,
        out_shape=jax.ShapeDtypeStruct((inputs.shape[0], 1), jnp.float32),
    )(inputs, acc, gamma, beta, wt, bias)


def kernel(inputs, tables, gamma, beta, W, bias):
    acc = _sc_gather_sum(inputs.reshape(-1), tables)
    wt = W.reshape(1, F_DN)
    return _tc_combine(inputs, acc, gamma, beta, wt, bias)


# speedup vs baseline: 7.1621x; 7.1621x over previous
import functools

import jax
import jax.numpy as jnp
from jax import lax
from jax.experimental import pallas as pl
from jax.experimental.pallas import tpu as pltpu
from jax.experimental.pallas import tpu_sc as plsc

F_SP = 26
F_DN = 13
NF = F_SP + F_DN
D = 16
V = 100000
VPAD = 100352
EPS = 1e-5

NC = 2
NS = 16
NW = NC * NS
CHUNK = 128
NG = CHUNK // 16

HALF_V = VPAD // 2
NROW16 = F_SP * VPAD // 16


def _tc_rowsum(tables_t):
    def body(in_ref, out_ref):
        out_ref[...] = jnp.sum(in_ref[0], axis=0)

    return pl.pallas_call(
        body,
        grid=(F_SP, 2),
        in_specs=[pl.BlockSpec((1, D, HALF_V), lambda f, h: (f, 0, h))],
        out_specs=pl.BlockSpec((HALF_V,), lambda f, h: (f * 2 + h,)),
        out_shape=jax.ShapeDtypeStruct((F_SP * VPAD,), jnp.float32),
    )(tables_t)


def _sc_gather_sum(inputs_flat, s16):
    b = inputs_flat.shape[0] // NF
    ngrp = b // CHUNK
    nchunk = ngrp // NW
    mesh = plsc.VectorSubcoreMesh(core_axis_name="c", subcore_axis_name="s")

    @functools.partial(
        pl.kernel,
        out_type=jax.ShapeDtypeStruct((ngrp, CHUNK), jnp.float32),
        mesh=mesh,
        scratch_types=[
            pltpu.VMEM((CHUNK * NF,), jnp.float32),
            pltpu.VMEM((F_SP, 1, CHUNK), jnp.int32),
            pltpu.VMEM((F_SP, CHUNK), jnp.int32),
            pltpu.VMEM((F_SP * CHUNK, D), jnp.float32),
            pltpu.VMEM((1, CHUNK), jnp.float32),
            pltpu.SemaphoreType.DMA,
        ],
        compiler_params=pltpu.CompilerParams(
            use_tc_tiling_on_sc=False, needs_layout_passes=False
        ),
    )
    def k(in_hbm, s_hbm, out_hbm, raw_v, idx_v, off_v, rows_v, red_v, gsem):
        wid = lax.axis_index("c") * NS + lax.axis_index("s")
        lane = lax.iota(jnp.int32, 16)

        @pl.loop(0, nchunk)
        def _chunk(kc):
            grp = wid * nchunk + kc
            pltpu.sync_copy(in_hbm.at[pl.ds(grp * CHUNK * NF, CHUNK * NF)], raw_v)

            @pl.loop(0, F_SP)
            def _field(f):
                for g in range(NG):
                    pos = lane * NF + (g * 16 * NF) + f
                    ids = plsc.load_gather(raw_v, [pos]).astype(jnp.int32)
                    p = ids + f * VPAD
                    idx_v[f, 0, pl.ds(g * 16, 16)] = p >> 4
                    off_v[f, pl.ds(g * 16, 16)] = p & 15
                pltpu.async_copy(
                    s_hbm.at[idx_v.at[f, 0]],
                    rows_v.at[pl.ds(f * CHUNK, CHUNK), :],
                    gsem,
                )

            pltpu.make_async_copy(
                s_hbm.at[pl.ds(0, F_SP * CHUNK), :], rows_v, gsem
            ).wait()

            for g in range(NG):
                rbase = lane + g * 16
                acc = plsc.load_gather(
                    rows_v, [rbase, off_v[0, pl.ds(g * 16, 16)]]
                )
                for f in range(1, F_SP):
                    acc = acc + plsc.load_gather(
                        rows_v, [rbase + f * CHUNK, off_v[f, pl.ds(g * 16, 16)]]
                    )
                red_v[0, pl.ds(g * 16, 16)] = acc

            pltpu.sync_copy(red_v, out_hbm.at[pl.ds(grp, 1), :])

    return k(inputs_flat, s16)


def _tc_combine(inputs, sp, gamma, beta, wt, bias):
    b = inputs.shape[0]

    def body(in_ref, sp_ref, g_ref, b_ref, w_ref, bias_ref, out_ref):
        d = in_ref[:, F_SP:]
        mean = jnp.mean(d, axis=0, keepdims=True)
        c = d - mean
        var = jnp.mean(c * c, axis=0, keepdims=True)
        bn = c * lax.rsqrt(var + EPS) * g_ref[...][None, :] + b_ref[...][None, :]
        dense_logit = jnp.sum(bn * w_ref[...], axis=1, keepdims=True)
        out_ref[...] = sp_ref[...] + dense_logit + bias_ref[...][None, :]

    return pl.pallas_call(
        body,
        out_shape=jax.ShapeDtypeStruct((b, 1), jnp.float32),
    )(inputs, sp, gamma, beta, wt, bias)


def kernel(inputs, tables, gamma, beta, W, bias):
    s_flat = _tc_rowsum(jnp.transpose(tables, (0, 2, 1)))
    s16 = s_flat.reshape(NROW16, D)
    sp = _sc_gather_sum(inputs.reshape(-1), s16)
    wt = W.reshape(1, F_DN)
    return _tc_combine(inputs, sp.reshape(inputs.shape[0], 1), gamma, beta, wt, bias)
